# Initial kernel scaffold; baseline (speedup 1.0000x reference)
import jax
import jax.numpy as jnp
from jax.experimental import pallas as pl


def kernel(x, edge_index, edge_attr, W1, b1, W2, b2, Wg, bg, We, att_src, att_dst, att_edge):
    raise NotImplementedError("write your pallas kernel here")


# speedup vs baseline: 10.9846x; 10.9846x over previous
import functools

import jax
import jax.numpy as jnp
from jax import lax
from jax.experimental import pallas as pl
from jax.experimental.pallas import tpu as pltpu
from jax.experimental.pallas import tpu_sc as plsc

NC = 2
NS = 16
NW = NC * NS
CE = 128
L = 16


def _mesh():
  return plsc.VectorSubcoreMesh(core_axis_name="c", subcore_axis_name="s")


def _deg_call(npad, chunks):
  rps = npad // NS

  @functools.partial(
      pl.kernel,
      out_type=jax.ShapeDtypeStruct((NC, npad, 16), jnp.float32),
      mesh=_mesh(),
      compiler_params=pltpu.CompilerParams(use_tc_tiling_on_sc=False),
      scratch_types=[
          pltpu.VMEM((chunks, CE), jnp.int32),
          pltpu.VMEM((CE, 16), jnp.float32),
          pltpu.VMEM_SHARED((npad, 16), jnp.float32),
      ],
  )
  def k(zd_hbm, one_hbm, dst_hbm, out_hbm, didx, ones, accd):
    c = lax.axis_index("c")
    s = lax.axis_index("s")
    wid = s * NC + c
    pltpu.sync_copy(zd_hbm.at[pl.ds(s * rps, rps)], accd.at[pl.ds(s * rps, rps)])
    pltpu.sync_copy(dst_hbm.at[wid], didx)
    pltpu.sync_copy(one_hbm, ones)
    plsc.subcore_barrier()

    def body(i, carry):
      pltpu.sync_copy(ones, accd.at[didx.at[i]], add=True)
      return carry

    lax.fori_loop(0, chunks, body, 0)
    plsc.subcore_barrier()
    pltpu.sync_copy(accd.at[pl.ds(s * rps, rps)],
                    out_hbm.at[c, pl.ds(s * rps, rps)])

  return k


def _seg_sum_call(npad, chunks):
  rps = npad // NS

  @functools.partial(
      pl.kernel,
      out_type=jax.ShapeDtypeStruct((NC, npad, 128), jnp.float32),
      mesh=_mesh(),
      scratch_types=[
          pltpu.VMEM((chunks, CE), jnp.int32),
          pltpu.VMEM((chunks, CE), jnp.int32),
          pltpu.VMEM((CE, 128), jnp.float32),
          pltpu.VMEM_SHARED((npad, 128), jnp.float32),
          pltpu.SemaphoreType.DMA,
      ],
  )
  def k(y_hbm, z_hbm, src_hbm, dst_hbm, out_hbm, sidx, didx, rows, acc, sem):
    c = lax.axis_index("c")
    s = lax.axis_index("s")
    wid = s * NC + c
    pltpu.sync_copy(z_hbm.at[pl.ds(s * rps, rps)], acc.at[pl.ds(s * rps, rps)])
    pltpu.sync_copy(src_hbm.at[wid], sidx)
    pltpu.sync_copy(dst_hbm.at[wid], didx)
    plsc.subcore_barrier()

    def body(i, carry):
      pltpu.async_copy(y_hbm.at[sidx.at[i]], rows, sem).wait()
      pltpu.sync_copy(rows, acc.at[didx.at[i]], add=True)
      return carry

    lax.fori_loop(0, chunks, body, 0)
    plsc.subcore_barrier()
    pltpu.sync_copy(acc.at[pl.ds(s * rps, rps)],
                    out_hbm.at[c, pl.ds(s * rps, rps)])

  return k


def _gat_den_call(npad, chunks):
  rps = npad // NS

  @functools.partial(
      pl.kernel,
      out_type=jax.ShapeDtypeStruct((NC, npad, 16), jnp.float32),
      mesh=_mesh(),
      compiler_params=pltpu.CompilerParams(use_tc_tiling_on_sc=False),
      scratch_types=[
          pltpu.VMEM((chunks, CE), jnp.int32),
          pltpu.VMEM((chunks, CE), jnp.int32),
          pltpu.VMEM((16,), jnp.float32),
          pltpu.VMEM((CE, 16), jnp.float32),
          pltpu.VMEM((CE, 16), jnp.float32),
          pltpu.VMEM((CE, 16), jnp.float32),
          pltpu.VMEM((CE, 16), jnp.float32),
          pltpu.VMEM_SHARED((npad, 16), jnp.float32),
          pltpu.SemaphoreType.DMA,
      ],
  )
  def k(zd_hbm, src_hbm, dst_hbm, ae_hbm, asn_hbm, adn_hbm, avec_hbm,
        outd_hbm, sidx, didx, avecv, asr, adr, aer, exrows, accd, sem):
    c = lax.axis_index("c")
    s = lax.axis_index("s")
    wid = s * NC + c
    pltpu.sync_copy(zd_hbm.at[pl.ds(s * rps, rps)], accd.at[pl.ds(s * rps, rps)])
    pltpu.sync_copy(src_hbm.at[wid], sidx)
    pltpu.sync_copy(dst_hbm.at[wid], didx)
    pltpu.sync_copy(avec_hbm, avecv)
    plsc.subcore_barrier()
    A = avecv[...]

    def chunk_body(i, carry):
      c2 = pltpu.async_copy(asn_hbm.at[sidx.at[i]], asr, sem)
      c3 = pltpu.async_copy(adn_hbm.at[didx.at[i]], adr, sem)
      pltpu.sync_copy(ae_hbm.at[wid, i], aer)
      c2.wait()
      c3.wait()

      def edge_body(j, cc):
        al = asr[j, :] + adr[j, :] + aer[j, :]
        al = jnp.maximum(al, 0.2 * al)
        exrows[j, :] = jnp.exp(al - A)
        return cc

      lax.fori_loop(0, CE, edge_body, 0)
      pltpu.sync_copy(exrows, accd.at[didx.at[i]], add=True)
      return carry

    lax.fori_loop(0, chunks, chunk_body, 0)
    plsc.subcore_barrier()
    pltpu.sync_copy(accd.at[pl.ds(s * rps, rps)],
                    outd_hbm.at[c, pl.ds(s * rps, rps)])

  return k


def _gat_num_call(npad, chunks):
  rps = npad // NS

  @functools.partial(
      pl.kernel,
      out_type=jax.ShapeDtypeStruct((NC, npad, 128), jnp.float32),
      mesh=_mesh(),
      compiler_params=pltpu.CompilerParams(use_tc_tiling_on_sc=False),
      scratch_types=[
          pltpu.VMEM((chunks, CE), jnp.int32),
          pltpu.VMEM((chunks, CE), jnp.int32),
          pltpu.VMEM((16,), jnp.float32),
          pltpu.VMEM((CE, 128), jnp.float32),
          pltpu.VMEM((CE, 16), jnp.float32),
          pltpu.VMEM((CE, 16), jnp.float32),
          pltpu.VMEM((CE, 16), jnp.float32),
          pltpu.VMEM_SHARED((npad, 128), jnp.float32),
          pltpu.SemaphoreType.DMA,
      ],
  )
  def k(xs_hbm, zn_hbm, src_hbm, dst_hbm, ae_hbm, asn_hbm, adn_hbm,
        avec_hbm, outn_hbm, sidx, didx, avecv, rows, asr, adr, aer, accn,
        sem):
    c = lax.axis_index("c")
    s = lax.axis_index("s")
    wid = s * NC + c
    pltpu.sync_copy(zn_hbm.at[pl.ds(s * rps, rps)], accn.at[pl.ds(s * rps, rps)])
    pltpu.sync_copy(src_hbm.at[wid], sidx)
    pltpu.sync_copy(dst_hbm.at[wid], didx)
    pltpu.sync_copy(avec_hbm, avecv)
    plsc.subcore_barrier()
    A = avecv[...]

    def chunk_body(i, carry):
      c1 = pltpu.async_copy(xs_hbm.at[sidx.at[i]], rows, sem)
      c2 = pltpu.async_copy(asn_hbm.at[sidx.at[i]], asr, sem)
      c3 = pltpu.async_copy(adn_hbm.at[didx.at[i]], adr, sem)
      pltpu.sync_copy(ae_hbm.at[wid, i], aer)
      c1.wait()
      c2.wait()
      c3.wait()

      def edge_body(j, cc):
        al = asr[j, :] + adr[j, :] + aer[j, :]
        al = jnp.maximum(al, 0.2 * al)
        exr = jnp.exp(al - A)
        for r in range(128 // L):
          rsl = pl.ds(r * L, L)
          rows[j, rsl] = rows[j, rsl] * exr
        return cc

      lax.fori_loop(0, CE, edge_body, 0)
      pltpu.sync_copy(rows, accn.at[didx.at[i]], add=True)
      return carry

    lax.fori_loop(0, chunks, chunk_body, 0)
    plsc.subcore_barrier()
    pltpu.sync_copy(accn.at[pl.ds(s * rps, rps)],
                    outn_hbm.at[c, pl.ds(s * rps, rps)])

  return k


def kernel(x, edge_index, edge_attr, W1, b1, W2, b2, Wg, bg, We,
           att_src, att_dst, att_edge):
  n = x.shape[0]
  e = edge_index.shape[1]
  src = edge_index[0].astype(jnp.int32)
  dst = edge_index[1].astype(jnp.int32)

  npad = -(-(n + 1) // 128) * 128
  chunks = -(-e // (NW * CE))
  epad = NW * chunks * CE
  srcp = jnp.full((epad,), n, jnp.int32).at[:e].set(src).reshape(NW, chunks, CE)
  dstp = jnp.full((epad,), n, jnp.int32).at[:e].set(dst).reshape(NW, chunks, CE)
  z = jnp.zeros((npad, 128), jnp.float32)
  zd = jnp.zeros((npad, 16), jnp.float32)
  pad_rows = jnp.zeros((npad - n, 128), jnp.float32)

  degp = _deg_call(npad, chunks)(zd, jnp.ones((CE, 16), jnp.float32), dstp)
  deg = degp[0, :n, 0] + degp[1, :n, 0] + 1.0
  dinv = lax.rsqrt(jnp.maximum(deg, 1.0))

  def gcn(h, W, b):
    y = (h @ W) * dinv[:, None]
    ypad = jnp.concatenate([y, pad_rows], 0)
    parts = _seg_sum_call(npad, chunks)(ypad, z, srcp, dstp)
    agg = parts[0, :n] + parts[1, :n] + y
    return jax.nn.relu(agg * dinv[:, None] + b)

  h1 = gcn(x, W1, b1)
  h2 = gcn(h1, W2, b2)

  xs = h2 @ Wg
  we = We @ att_edge
  ae = edge_attr @ we
  ae_loop = jnp.mean(edge_attr, axis=0) @ we
  asn = xs @ att_src
  adn = xs @ att_dst
  lrelu = lambda v: jnp.maximum(v, 0.2 * v)
  A = lrelu(jnp.max(asn) + jnp.max(adn) + jnp.maximum(jnp.max(ae), ae_loop))
  avec = jnp.full((16,), A, jnp.float32)

  xsp = jnp.concatenate([xs, pad_rows], 0)
  asnp = jnp.concatenate([asn, jnp.zeros((npad - n,), jnp.float32)], 0)
  adnp = jnp.concatenate([adn, jnp.zeros((npad - n,), jnp.float32)], 0)
  asn16 = jnp.broadcast_to(asnp[:, None], (npad, 16))
  adn16 = jnp.broadcast_to(adnp[:, None], (npad, 16))
  aepf = jnp.zeros((epad,), jnp.float32).at[:e].set(ae)
  ae16 = jnp.broadcast_to(aepf[:, None], (epad, 16)).reshape(NW, chunks, CE, 16)

  dparts = _gat_den_call(npad, chunks)(
      zd, srcp, dstp, ae16, asn16, adn16, avec)
  nparts = _gat_num_call(npad, chunks)(
      xsp, z, srcp, dstp, ae16, asn16, adn16, avec)

  ex_l = jnp.exp(lrelu(asn + adn + ae_loop) - A)
  den = dparts[0, :n, 0] + dparts[1, :n, 0] + ex_l
  num = nparts[0, :n] + nparts[1, :n] + xs * ex_l[:, None]
  g = num / den[:, None] + bg

  return jnp.stack([h1, h2, g], axis=0)
